# baseline (speedup 1.0000x reference)
import jax
import jax.numpy as jnp
from jax.experimental import pallas as pl


def _maxes_body(x_ref, out_ref):
    i = pl.program_id(0)
    x = x_ref[...]
    N, m, _ = x.shape
    C2 = out_ref.shape[1]
    C = C2 // 2
    per_row = C // N
    row = jax.lax.broadcasted_iota(jnp.int32, (m, m), 0)
    col = jax.lax.broadcasted_iota(jnp.int32, (m, m), 1)
    eq = (row == col)[None]
    neg = jnp.float32(-jnp.inf)
    dmax = jnp.max(jnp.where(eq, x, neg), axis=(1, 2)).reshape(1, N)
    omax = jnp.max(jnp.where(eq, neg, x), axis=(1, 2)).reshape(1, N)
    n_steps = pl.num_programs(0)
    for step in range(n_steps):
        b = step // per_row
        c0 = (step % per_row) * N

        @pl.when(i == step)
        def _(b=b, c0=c0):
            out_ref[b : b + 1, c0 : c0 + N] = dmax
            out_ref[b : b + 1, C + c0 : C + c0 + N] = omax


def kernel(x):
    B, C, m, _ = x.shape
    n_mat = B * C
    N = 16
    return pl.pallas_call(
        _maxes_body,
        grid=(n_mat // N,),
        in_specs=[pl.BlockSpec((N, m, m), lambda i: (i, 0, 0))],
        out_specs=pl.BlockSpec((B, 2 * C), lambda i: (0, 0)),
        out_shape=jax.ShapeDtypeStruct((B, 2 * C), x.dtype),
    )(x.reshape(n_mat, m, m))


# speedup vs baseline: 1.0266x; 1.0266x over previous
import jax
import jax.numpy as jnp
from jax.experimental import pallas as pl


def _maxes_body(x_ref, out_ref):
    i = pl.program_id(0)
    x = x_ref[...]
    N, m, _ = x.shape
    C2 = out_ref.shape[1]
    C = C2 // 2
    per_row = C // N
    row = jax.lax.broadcasted_iota(jnp.int32, (m, m), 0)
    col = jax.lax.broadcasted_iota(jnp.int32, (m, m), 1)
    eq = (row == col)[None]
    neg = jnp.float32(-jnp.inf)
    dmax = jnp.max(jnp.where(eq, x, neg), axis=(1, 2)).reshape(1, N)
    omax = jnp.max(jnp.where(eq, neg, x), axis=(1, 2)).reshape(1, N)
    n_steps = pl.num_programs(0)
    for step in range(n_steps):
        b = step // per_row
        c0 = (step % per_row) * N

        @pl.when(i == step)
        def _(b=b, c0=c0):
            out_ref[b : b + 1, c0 : c0 + N] = dmax
            out_ref[b : b + 1, C + c0 : C + c0 + N] = omax


def kernel(x):
    B, C, m, _ = x.shape
    n_mat = B * C
    N = 8
    return pl.pallas_call(
        _maxes_body,
        grid=(n_mat // N,),
        in_specs=[pl.BlockSpec((N, m, m), lambda i: (i, 0, 0))],
        out_specs=pl.BlockSpec((B, 2 * C), lambda i: (0, 0)),
        out_shape=jax.ShapeDtypeStruct((B, 2 * C), x.dtype),
    )(x.reshape(n_mat, m, m))
